# baseline (speedup 1.0000x reference)
import jax
import jax.numpy as jnp
from jax import lax
from jax.experimental import pallas as pl
from jax.experimental.pallas import tpu as pltpu
from jax.experimental.pallas import tpu_sc as plsc

_B = 16384
_D = 32
_W = 128
_NC = 2
_NS = 16
_NW = _NC * _NS
_BPW = _B // _NW
_CH = 128
_NCHUNK = _BPW // _CH
_ORPW = _BPW * _D // _W


def _emb_body(table_hbm, idx_hbm, out_hbm,
              idx_v, gidx_v, buf0, buf1, out_v, sem0, sem1):
    wid = lax.axis_index("s") * _NC + lax.axis_index("c")
    base = wid * _BPW

    pltpu.sync_copy(idx_hbm.at[pl.ds(base, _BPW)], idx_v)
    for i in range(_BPW // 16):
        v = idx_v[pl.ds(i * 16, 16)]
        gidx_v[i // (_CH // 16), pl.ds((i % (_CH // 16)) * 16, 16)] = (
            lax.shift_right_logical(v, 2))

    bufs = (buf0, buf1)
    sems = (sem0, sem1)

    def fire(c):
        return pltpu.async_copy(table_hbm.at[gidx_v.at[c]], bufs[c % 2],
                                sems[c % 2])

    def extract(c):
        buf = bufs[c % 2]

        def block(b, carry):
            iv = idx_v[pl.ds(c * _CH + b * 16, 16)]
            for i in range(16):
                off = lax.bitwise_and(iv[i], jnp.int32(3)) * 32
                jloc = b * 16 + i
                orow = c * (_CH // 4) + b * 4 + i // 4
                ocol = (i % 4) * 32
                out_v[orow, pl.ds(ocol, 16)] = buf[jloc, pl.ds(off, 16)]
                out_v[orow, pl.ds(ocol + 16, 16)] = (
                    buf[jloc, pl.ds(off + 16, 16)])
            return carry

        lax.fori_loop(0, _CH // 16, block, 0)

    handles = [fire(0), fire(1)]
    for c in range(_NCHUNK):
        handles[c % 2].wait()
        extract(c)
        if c + 2 < _NCHUNK:
            handles[c % 2] = fire(c + 2)

    pltpu.sync_copy(out_v, out_hbm.at[pl.ds(wid * _ORPW, _ORPW)])


@jax.jit
def _lookup(table4, idx):
    mesh = plsc.VectorSubcoreMesh(core_axis_name="c", subcore_axis_name="s")
    f = pl.kernel(
        _emb_body,
        out_type=jax.ShapeDtypeStruct((_B * _D // _W, _W), jnp.float32),
        mesh=mesh,
        scratch_types=[
            pltpu.VMEM((_BPW,), jnp.int32),
            pltpu.VMEM((_NCHUNK, _CH), jnp.int32),
            pltpu.VMEM((_CH, _W), jnp.float32),
            pltpu.VMEM((_CH, _W), jnp.float32),
            pltpu.VMEM((_ORPW, _W), jnp.float32),
            pltpu.SemaphoreType.DMA,
            pltpu.SemaphoreType.DMA,
        ],
    )
    return f(table4, idx)


def kernel(idx, table):
    table4 = table.reshape(-1, _W)
    out = _lookup(table4, idx.astype(jnp.int32))
    return out.reshape(-1, _D, 1, 1)


# speedup vs baseline: 1.6430x; 1.6430x over previous
import jax
import jax.numpy as jnp
from jax import lax
from jax.experimental import pallas as pl
from jax.experimental.pallas import tpu as pltpu
from jax.experimental.pallas import tpu_sc as plsc

_B = 16384
_D = 32
_NC = 2
_NS = 16
_NW = _NC * _NS
_BPW = _B // _NW
_CH = 64
_NCHUNK = _BPW // _CH


def _emb_body(table_hbm, idx_hbm, out_hbm, idx_v, out_v, sem0, sem1):
    wid = lax.axis_index("s") * _NC + lax.axis_index("c")
    base = wid * _BPW
    pltpu.sync_copy(idx_hbm.at[pl.ds(base, _BPW)], idx_v)
    sems = (sem0, sem1)

    def issue_chunk(q):
        hs = []
        for bb in range(_CH // 16):
            iv = idx_v[pl.ds(q * _CH + bb * 16, 16)]
            for i in range(16):
                j = q * _CH + bb * 16 + i
                hs.append(pltpu.async_copy(table_hbm.at[iv[i]],
                                           out_v.at[j], sems[q % 2]))
        return hs

    pending = {}
    for q in range(_NCHUNK):
        pending[q] = issue_chunk(q)
        if q >= 1:
            for h in pending.pop(q - 1):
                h.wait()
    for h in pending.pop(_NCHUNK - 1):
        h.wait()

    pltpu.sync_copy(out_v, out_hbm.at[pl.ds(base, _BPW)])


@jax.jit
def _lookup(table, idx):
    mesh = plsc.VectorSubcoreMesh(core_axis_name="c", subcore_axis_name="s")
    f = pl.kernel(
        _emb_body,
        out_type=jax.ShapeDtypeStruct((_B, _D), jnp.float32),
        mesh=mesh,
        scratch_types=[
            pltpu.VMEM((_BPW,), jnp.int32),
            pltpu.VMEM((_BPW, _D), jnp.float32),
            pltpu.SemaphoreType.DMA,
            pltpu.SemaphoreType.DMA,
        ],
    )
    return f(table, idx)


def kernel(idx, table):
    out = _lookup(table, idx.astype(jnp.int32))
    return out.reshape(-1, _D, 1, 1)
